# baseline (speedup 1.0000x reference)
import functools

import jax
import jax.numpy as jnp
from jax import lax
from jax.experimental import pallas as pl
from jax.experimental.pallas import tpu as pltpu
from jax.experimental.pallas import tpu_sc as plsc

_SHAPE = (2, 32, 16384, 20)
_M = _SHAPE[0] * _SHAPE[1] * _SHAPE[2]
_NC, _NS = 2, 16
_NW = _NC * _NS
_RPW = _M // _NW
_R = 2048
_CHUNKS = _RPW // _R


def _make_copy_kernel(row):
    mesh = plsc.VectorSubcoreMesh(core_axis_name="c", subcore_axis_name="s")

    @functools.partial(
        pl.kernel,
        mesh=mesh,
        out_type=jax.ShapeDtypeStruct((_M, 1, row), jnp.int32),
    )
    def copy_kernel(x_hbm, o_hbm):
        wid = lax.axis_index("s") * _NC + lax.axis_index("c")
        base = wid * _RPW
        pltpu.sync_copy(
            x_hbm.at[pl.ds(base, _RPW), pl.ds(0, 1)],
            o_hbm.at[pl.ds(base, _RPW)],
        )

    return copy_kernel


def kernel(edge_index):
    num = _SHAPE[3] // 2
    if edge_index.dtype == jnp.int64:
        words = lax.bitcast_convert_type(edge_index, jnp.int32)
        xi = words.reshape(_M, 2, num * 2)
        out = _make_copy_kernel(num * 2)(xi)
        out = out.reshape(_SHAPE[0], _SHAPE[1], _SHAPE[2], num, 2)
        return lax.bitcast_convert_type(out, jnp.int64)
    xi = edge_index.reshape(_M, 2, num)
    out = _make_copy_kernel(num)(xi)
    return out.reshape(_SHAPE[0], _SHAPE[1], _SHAPE[2], num).astype(edge_index.dtype)
